# Initial kernel scaffold; baseline (speedup 1.0000x reference)
import jax
import jax.numpy as jnp
from jax.experimental import pallas as pl


def kernel(x, edge_index, edge_weight, W1, b1, W2, b2, Wout, bout):
    raise NotImplementedError("write your pallas kernel here")


# speedup vs baseline: 8.8881x; 8.8881x over previous
import functools

import jax
import jax.numpy as jnp
from jax import lax
from jax.experimental import pallas as pl
from jax.experimental.pallas import tpu as pltpu
from jax.experimental.pallas import tpu_sc as plsc

N = 10000
NP = 10240
D = 128
E = 320000
EP = 327680
ODIM = 40
INV_T = 5.0

NC = 2
NS = 16
NWORK = NC * NS
EPW = EP // NWORK
ROWS_PW = EPW // D
NCHUNK = 5
CROWS = ROWS_PW // NCHUNK
NPW = NP // NS
NB = 2048
NGRID = NP // NB

_sc_mesh = plsc.VectorSubcoreMesh(core_axis_name="c", subcore_axis_name="s")


@functools.partial(
    pl.kernel,
    out_type=(
        jax.ShapeDtypeStruct((NP,), jnp.float32),
        jax.ShapeDtypeStruct((NP,), jnp.float32),
    ),
    mesh=_sc_mesh,
    compiler_params=pltpu.CompilerParams(needs_layout_passes=False),
    scratch_types=[
        pltpu.VMEM((CROWS, D), jnp.int32),
        pltpu.VMEM((CROWS, D), jnp.float32),
        pltpu.VMEM((NP,), jnp.float32),
        pltpu.VMEM_SHARED((NS, NP), jnp.float32),
        pltpu.VMEM((NS, NPW), jnp.float32),
        pltpu.VMEM((NPW,), jnp.float32),
    ],
)
def _sc_degree(col2d, ew2d, deg0_out, deg1_out,
               colbuf, ewbuf, dloc, sdeg, vbuf, dsum):
    cid = lax.axis_index("c")
    sid = lax.axis_index("s")
    wid = cid * NS + sid

    def zbody(i, _):
        dloc[pl.ds(i * 16, 16)] = jnp.zeros((16,), jnp.float32)
        return 0
    lax.fori_loop(0, NP // 16, zbody, 0)

    for c in range(NCHUNK):
        rb = wid * ROWS_PW + c * CROWS
        pltpu.sync_copy(col2d.at[pl.ds(rb, CROWS)], colbuf)
        pltpu.sync_copy(ew2d.at[pl.ds(rb, CROWS)], ewbuf)

        def rbody(r, _):
            def kbody(k, _2):
                idx = colbuf[r, pl.ds(k * 16, 16)]
                w = ewbuf[r, pl.ds(k * 16, 16)]
                plsc.addupdate_scatter(dloc, [idx], w)
                return 0
            lax.fori_loop(0, D // 16, kbody, 0)
            return 0
        lax.fori_loop(0, CROWS, rbody, 0)

    pltpu.sync_copy(dloc, sdeg.at[sid])
    plsc.subcore_barrier()
    pltpu.sync_copy(sdeg.at[:, pl.ds(sid * NPW, NPW)], vbuf)

    def gbody(g, _):
        acc = vbuf[0, pl.ds(g * 16, 16)]
        for t in range(1, NS):
            acc = acc + vbuf[t, pl.ds(g * 16, 16)]
        dsum[pl.ds(g * 16, 16)] = acc
        return 0
    lax.fori_loop(0, NPW // 16, gbody, 0)

    @pl.when(cid == 0)
    def _():
        pltpu.sync_copy(dsum, deg0_out.at[pl.ds(sid * NPW, NPW)])

    @pl.when(cid == 1)
    def _():
        pltpu.sync_copy(dsum, deg1_out.at[pl.ds(sid * NPW, NPW)])


NSUB = EPW // D
MROWS = 8
NSUPER = NSUB // MROWS

_SPLAT_DNUMS = lax.GatherDimensionNumbers(
    offset_dims=(), collapsed_slice_dims=(0,), start_index_map=(0,))


def _splat(v, lane):
    idx = jnp.full((16,), lane, jnp.int32)
    return lax.gather(v, idx[:, None], _SPLAT_DNUMS, (1,),
                      mode=lax.GatherScatterMode.PROMISE_IN_BOUNDS)


@functools.partial(
    pl.kernel,
    out_type=jax.ShapeDtypeStruct((NC, NP, D), jnp.float32),
    mesh=_sc_mesh,
    scratch_types=[
        pltpu.VMEM((2, MROWS, D), jnp.int32),
        pltpu.VMEM((2, MROWS, D), jnp.int32),
        pltpu.VMEM((2, MROWS, D), jnp.float32),
        pltpu.VMEM((D, D), jnp.float32),
        pltpu.VMEM((D, D), jnp.float32),
        pltpu.VMEM_SHARED((NP, D), jnp.float32),
        pltpu.SemaphoreType.DMA,
        pltpu.SemaphoreType.DMA,
    ],
)
def _sc_spmm(hs, row2d, col2d, ew2d, zeros, s_out,
             rowb, colb, ewb, rbuf0, rbuf1, acc, sg0, sg1):
    cid = lax.axis_index("c")
    sid = lax.axis_index("s")
    wid = cid * NS + sid
    mbase = wid * ROWS_PW

    pltpu.sync_copy(zeros.at[pl.ds(sid * NPW, NPW)], acc.at[pl.ds(sid * NPW, NPW)])
    plsc.subcore_barrier()

    pltpu.sync_copy(row2d.at[pl.ds(mbase, MROWS)], rowb.at[0])
    pltpu.sync_copy(col2d.at[pl.ds(mbase, MROWS)], colb.at[0])
    pltpu.sync_copy(ew2d.at[pl.ds(mbase, MROWS)], ewb.at[0])
    pltpu.async_copy(hs.at[rowb.at[0, 0]], rbuf0, sg0)

    def _step(t, rb, gs, orb, ogs):
        m = (t >> 3) & 1
        j = t & 7
        pltpu.make_async_copy(hs.at[rowb.at[m, j]], rb, gs).wait()

        @pl.when((j == 7) & (t + 1 < NSUB))
        def _():
            s1 = (t + 1) >> 3
            mq = s1 & 1
            rbase = mbase + s1 * MROWS
            pltpu.sync_copy(row2d.at[pl.ds(rbase, MROWS)], rowb.at[mq])
            pltpu.sync_copy(col2d.at[pl.ds(rbase, MROWS)], colb.at[mq])
            pltpu.sync_copy(ew2d.at[pl.ds(rbase, MROWS)], ewb.at[mq])

        @pl.when(t + 1 < NSUB)
        def _():
            t1 = t + 1
            pltpu.async_copy(hs.at[rowb.at[(t1 >> 3) & 1, t1 & 7]], orb, ogs)

        def gbody(g16, _):
            wv = ewb[m, j, pl.ds(g16 * 16, 16)]

            def lbody(l, _2):
                e = g16 * 16 + l
                wl = _splat(wv, l)
                for g in range(D // 16):
                    sl = pl.ds(g * 16, 16)
                    rb[e, sl] = rb[e, sl] * wl
                return 0
            lax.fori_loop(0, 16, lbody, 0)
            return 0
        lax.fori_loop(0, D // 16, gbody, 0)

        pltpu.sync_copy(rb, acc.at[colb.at[m, j]], add=True)

    def body(hp, _):
        _step(2 * hp, rbuf0, sg0, rbuf1, sg1)
        _step(2 * hp + 1, rbuf1, sg1, rbuf0, sg0)
        return 0
    lax.fori_loop(0, NSUB // 2, body, 0)

    plsc.subcore_barrier()
    pltpu.sync_copy(acc.at[pl.ds(sid * NPW, NPW)],
                    s_out.at[cid, pl.ds(sid * NPW, NPW)])


def _dis_block(d0_ref, d1_ref):
    i = pl.program_id(0)
    deg = d0_ref[pl.ds(i * NB, NB)] + d1_ref[pl.ds(i * NB, NB)] + 1.0
    return lax.rsqrt(deg)


def _tc_prep_body(x_ref, w_ref, d0_ref, d1_ref, hs_ref):
    dis = _dis_block(d0_ref, d1_ref)
    h = jnp.dot(x_ref[...], w_ref[...], preferred_element_type=jnp.float32)
    hs_ref[...] = h * dis[:, None]


def _tc_mid_body(s0_ref, s1_ref, hs_ref, d0_ref, d1_ref, w_ref, b_ref, out_ref):
    dis = _dis_block(d0_ref, d1_ref)
    t = (s0_ref[...] + s1_ref[...] + hs_ref[...]) * dis[:, None] + b_ref[...][None, :]
    r = jnp.maximum(t, 0.0)
    h2 = jnp.dot(r, w_ref[...], preferred_element_type=jnp.float32)
    out_ref[...] = h2 * dis[:, None]


def _tc_final_body(s0_ref, s1_ref, hs_ref, d0_ref, d1_ref, b_ref,
                   wout_ref, bout_ref, out_ref):
    dis = _dis_block(d0_ref, d1_ref)
    t = (s0_ref[...] + s1_ref[...] + hs_ref[...]) * dis[:, None] + b_ref[...][None, :]
    logits = jnp.dot(t, wout_ref[...], preferred_element_type=jnp.float32)
    z = (logits + bout_ref[...][None, :]) * INV_T
    m = jnp.max(z, axis=1, keepdims=True)
    zz = z - m
    lse = jnp.log(jnp.sum(jnp.exp(zz), axis=1, keepdims=True))
    out_ref[...] = zz - lse


def _blk(shape):
    return pl.BlockSpec(shape, lambda i: (i,) + (0,) * (len(shape) - 1))


def _full(shape):
    return pl.BlockSpec(shape, lambda i: (0,) * len(shape))


def _tc_prep(x, W1, d0, d1):
    return pl.pallas_call(
        _tc_prep_body,
        grid=(NGRID,),
        in_specs=[_blk((NB, D)), _full((D, D)), _full((NP,)), _full((NP,))],
        out_specs=_blk((NB, D)),
        out_shape=jax.ShapeDtypeStruct((NP, D), jnp.float32),
    )(x, W1, d0, d1)


def _tc_mid(s0, s1, hs, d0, d1, W2, b1):
    return pl.pallas_call(
        _tc_mid_body,
        grid=(NGRID,),
        in_specs=[_blk((NB, D)), _blk((NB, D)), _blk((NB, D)),
                  _full((NP,)), _full((NP,)), _full((D, D)), _full((D,))],
        out_specs=_blk((NB, D)),
        out_shape=jax.ShapeDtypeStruct((NP, D), jnp.float32),
    )(s0, s1, hs, d0, d1, W2, b1)


def _tc_final(s0, s1, hs, d0, d1, b2, Wout, bout):
    return pl.pallas_call(
        _tc_final_body,
        grid=(NGRID,),
        in_specs=[_blk((NB, D)), _blk((NB, D)), _blk((NB, D)),
                  _full((NP,)), _full((NP,)), _full((D,)),
                  _full((D, ODIM)), _full((ODIM,))],
        out_specs=_blk((NB, ODIM)),
        out_shape=jax.ShapeDtypeStruct((NP, ODIM), jnp.float32),
    )(s0, s1, hs, d0, d1, b2, Wout, bout)


def kernel(x, edge_index, edge_weight, W1, b1, W2, b2, Wout, bout):
    x = x.astype(jnp.float32)
    row = edge_index[0].astype(jnp.int32)
    col = edge_index[1].astype(jnp.int32)
    ew = edge_weight.astype(jnp.float32)

    pad = EP - E
    row2d = jnp.pad(row, (0, pad)).reshape(EP // D, D)
    col2d = jnp.pad(col, (0, pad)).reshape(EP // D, D)
    ew2d = jnp.pad(ew, (0, pad)).reshape(EP // D, D)

    xp = jnp.pad(x, ((0, NP - N), (0, 0)))
    zeros = jnp.zeros((NP, D), jnp.float32)

    d0, d1 = _sc_degree(col2d, ew2d)

    hs1 = _tc_prep(xp, W1, d0, d1)
    s1 = _sc_spmm(hs1, row2d, col2d, ew2d, zeros)
    hs2 = _tc_mid(s1[0], s1[1], hs1, d0, d1, W2, b1)
    s2 = _sc_spmm(hs2, row2d, col2d, ew2d, zeros)
    out = _tc_final(s2[0], s2[1], hs2, d0, d1, b2, Wout, bout)
    return out[:N]


# speedup vs baseline: 9.8812x; 1.1117x over previous
import functools

import jax
import jax.numpy as jnp
from jax import lax
from jax.experimental import pallas as pl
from jax.experimental.pallas import tpu as pltpu
from jax.experimental.pallas import tpu_sc as plsc

N = 10000
NP = 10240
D = 128
E = 320000
EP = 327680
ODIM = 40
INV_T = 5.0

NC = 2
NS = 16
NWORK = NC * NS
EPW = EP // NWORK
ROWS_PW = EPW // D
NCHUNK = 5
CROWS = ROWS_PW // NCHUNK
NPW = NP // NS
NB = 2048
NGRID = NP // NB

_sc_mesh = plsc.VectorSubcoreMesh(core_axis_name="c", subcore_axis_name="s")


@functools.partial(
    pl.kernel,
    out_type=(
        jax.ShapeDtypeStruct((NP,), jnp.float32),
        jax.ShapeDtypeStruct((NP,), jnp.float32),
    ),
    mesh=_sc_mesh,
    compiler_params=pltpu.CompilerParams(needs_layout_passes=False),
    scratch_types=[
        pltpu.VMEM((CROWS, D), jnp.int32),
        pltpu.VMEM((CROWS, D), jnp.float32),
        pltpu.VMEM((NP,), jnp.float32),
        pltpu.VMEM_SHARED((NS, NP), jnp.float32),
        pltpu.VMEM((NS, NPW), jnp.float32),
        pltpu.VMEM((NPW,), jnp.float32),
    ],
)
def _sc_degree(col2d, ew2d, deg0_out, deg1_out,
               colbuf, ewbuf, dloc, sdeg, vbuf, dsum):
    cid = lax.axis_index("c")
    sid = lax.axis_index("s")
    wid = cid * NS + sid

    def zbody(i, _):
        dloc[pl.ds(i * 16, 16)] = jnp.zeros((16,), jnp.float32)
        return 0
    lax.fori_loop(0, NP // 16, zbody, 0)

    for c in range(NCHUNK):
        rb = wid * ROWS_PW + c * CROWS
        pltpu.sync_copy(col2d.at[pl.ds(rb, CROWS)], colbuf)
        pltpu.sync_copy(ew2d.at[pl.ds(rb, CROWS)], ewbuf)

        def rbody(r, _):
            def kbody(k, _2):
                idx = colbuf[r, pl.ds(k * 16, 16)]
                w = ewbuf[r, pl.ds(k * 16, 16)]
                plsc.addupdate_scatter(dloc, [idx], w)
                return 0
            lax.fori_loop(0, D // 16, kbody, 0)
            return 0
        lax.fori_loop(0, CROWS, rbody, 0)

    pltpu.sync_copy(dloc, sdeg.at[sid])
    plsc.subcore_barrier()
    pltpu.sync_copy(sdeg.at[:, pl.ds(sid * NPW, NPW)], vbuf)

    def gbody(g, _):
        acc = vbuf[0, pl.ds(g * 16, 16)]
        for t in range(1, NS):
            acc = acc + vbuf[t, pl.ds(g * 16, 16)]
        dsum[pl.ds(g * 16, 16)] = acc
        return 0
    lax.fori_loop(0, NPW // 16, gbody, 0)

    @pl.when(cid == 0)
    def _():
        pltpu.sync_copy(dsum, deg0_out.at[pl.ds(sid * NPW, NPW)])

    @pl.when(cid == 1)
    def _():
        pltpu.sync_copy(dsum, deg1_out.at[pl.ds(sid * NPW, NPW)])


NSUB0 = 120
NSUB1 = 40
MROWS = 8
C1BASE = NS * NSUB0

_SPLAT_DNUMS = lax.GatherDimensionNumbers(
    offset_dims=(), collapsed_slice_dims=(0,), start_index_map=(0,))


def _splat(v, lane):
    idx = jnp.full((16,), lane, jnp.int32)
    return lax.gather(v, idx[:, None], _SPLAT_DNUMS, (1,),
                      mode=lax.GatherScatterMode.PROMISE_IN_BOUNDS)


@functools.partial(
    pl.kernel,
    out_type=jax.ShapeDtypeStruct((NC, NP, D), jnp.float32),
    mesh=_sc_mesh,
    scratch_types=[
        pltpu.VMEM((2, MROWS, D), jnp.int32),
        pltpu.VMEM((2, MROWS, D), jnp.int32),
        pltpu.VMEM((2, MROWS, D), jnp.float32),
        pltpu.VMEM((D, D), jnp.float32),
        pltpu.VMEM((D, D), jnp.float32),
        pltpu.VMEM_SHARED((NP, D), jnp.float32),
        pltpu.SemaphoreType.DMA,
        pltpu.SemaphoreType.DMA,
        pltpu.SemaphoreType.DMA,
        pltpu.SemaphoreType.DMA,
    ],
)
def _sc_spmm(hs, row2d, col2d, ew2d, zeros, s_out,
             rowb, colb, ewb, rbuf0, rbuf1, acc, sg0, sg1, ss0, ss1):
    cid = lax.axis_index("c")
    sid = lax.axis_index("s")
    nsub = jnp.where(cid == 0, NSUB0, NSUB1)
    mbase = jnp.where(cid == 0, sid * NSUB0, C1BASE + sid * NSUB1)

    pltpu.sync_copy(zeros.at[pl.ds(sid * NPW, NPW)], acc.at[pl.ds(sid * NPW, NPW)])
    plsc.subcore_barrier()

    pltpu.sync_copy(row2d.at[pl.ds(mbase, MROWS)], rowb.at[0])
    pltpu.sync_copy(col2d.at[pl.ds(mbase, MROWS)], colb.at[0])
    pltpu.sync_copy(ew2d.at[pl.ds(mbase, MROWS)], ewb.at[0])
    pltpu.async_copy(hs.at[rowb.at[0, 0]], rbuf0, sg0)

    def _step(t, rb, gs, ssem, orb, ogs, ossem):
        m = (t >> 3) & 1
        j = t & 7
        pltpu.make_async_copy(hs.at[rowb.at[m, j]], rb, gs).wait()

        @pl.when(t >= 1)
        def _():
            pltpu.make_async_copy(orb, acc.at[colb.at[0, 0]], ossem).wait()

        @pl.when((j == 7) & (t + 1 < nsub))
        def _():
            s1 = (t + 1) >> 3
            mq = s1 & 1
            rbase = mbase + s1 * MROWS
            pltpu.sync_copy(row2d.at[pl.ds(rbase, MROWS)], rowb.at[mq])
            pltpu.sync_copy(col2d.at[pl.ds(rbase, MROWS)], colb.at[mq])
            pltpu.sync_copy(ew2d.at[pl.ds(rbase, MROWS)], ewb.at[mq])

        @pl.when(t + 1 < nsub)
        def _():
            t1 = t + 1
            pltpu.async_copy(hs.at[rowb.at[(t1 >> 3) & 1, t1 & 7]], orb, ogs)

        def gbody(g16, _):
            wv = ewb[m, j, pl.ds(g16 * 16, 16)]

            def lbody(l, _2):
                e = g16 * 16 + l
                wl = _splat(wv, l)
                for g in range(D // 16):
                    sl = pl.ds(g * 16, 16)
                    rb[e, sl] = rb[e, sl] * wl
                return 0
            lax.fori_loop(0, 16, lbody, 0)
            return 0
        lax.fori_loop(0, D // 16, gbody, 0)

        pltpu.async_copy(rb, acc.at[colb.at[m, j]], ssem, add=True)

    def body(hp, _):
        _step(2 * hp, rbuf0, sg0, ss0, rbuf1, sg1, ss1)
        _step(2 * hp + 1, rbuf1, sg1, ss1, rbuf0, sg0, ss0)
        return 0
    lax.fori_loop(0, nsub // 2, body, 0)

    pltpu.make_async_copy(rbuf1, acc.at[colb.at[0, 0]], ss1).wait()

    plsc.subcore_barrier()
    pltpu.sync_copy(acc.at[pl.ds(sid * NPW, NPW)],
                    s_out.at[cid, pl.ds(sid * NPW, NPW)])


def _dis_block(d0_ref, d1_ref):
    i = pl.program_id(0)
    deg = d0_ref[pl.ds(i * NB, NB)] + d1_ref[pl.ds(i * NB, NB)] + 1.0
    return lax.rsqrt(deg)


def _tc_prep_body(x_ref, w_ref, d0_ref, d1_ref, hs_ref):
    dis = _dis_block(d0_ref, d1_ref)
    h = jnp.dot(x_ref[...], w_ref[...], preferred_element_type=jnp.float32)
    hs_ref[...] = h * dis[:, None]


def _tc_mid_body(s0_ref, s1_ref, hs_ref, d0_ref, d1_ref, w_ref, b_ref, out_ref):
    dis = _dis_block(d0_ref, d1_ref)
    t = (s0_ref[...] + s1_ref[...] + hs_ref[...]) * dis[:, None] + b_ref[...][None, :]
    r = jnp.maximum(t, 0.0)
    h2 = jnp.dot(r, w_ref[...], preferred_element_type=jnp.float32)
    out_ref[...] = h2 * dis[:, None]


def _tc_final_body(s0_ref, s1_ref, hs_ref, d0_ref, d1_ref, b_ref,
                   wout_ref, bout_ref, out_ref):
    dis = _dis_block(d0_ref, d1_ref)
    t = (s0_ref[...] + s1_ref[...] + hs_ref[...]) * dis[:, None] + b_ref[...][None, :]
    logits = jnp.dot(t, wout_ref[...], preferred_element_type=jnp.float32)
    z = (logits + bout_ref[...][None, :]) * INV_T
    m = jnp.max(z, axis=1, keepdims=True)
    zz = z - m
    lse = jnp.log(jnp.sum(jnp.exp(zz), axis=1, keepdims=True))
    out_ref[...] = zz - lse


def _blk(shape):
    return pl.BlockSpec(shape, lambda i: (i,) + (0,) * (len(shape) - 1))


def _full(shape):
    return pl.BlockSpec(shape, lambda i: (0,) * len(shape))


def _tc_prep(x, W1, d0, d1):
    return pl.pallas_call(
        _tc_prep_body,
        grid=(NGRID,),
        in_specs=[_blk((NB, D)), _full((D, D)), _full((NP,)), _full((NP,))],
        out_specs=_blk((NB, D)),
        out_shape=jax.ShapeDtypeStruct((NP, D), jnp.float32),
    )(x, W1, d0, d1)


def _tc_mid(s0, s1, hs, d0, d1, W2, b1):
    return pl.pallas_call(
        _tc_mid_body,
        grid=(NGRID,),
        in_specs=[_blk((NB, D)), _blk((NB, D)), _blk((NB, D)),
                  _full((NP,)), _full((NP,)), _full((D, D)), _full((D,))],
        out_specs=_blk((NB, D)),
        out_shape=jax.ShapeDtypeStruct((NP, D), jnp.float32),
    )(s0, s1, hs, d0, d1, W2, b1)


def _tc_final(s0, s1, hs, d0, d1, b2, Wout, bout):
    return pl.pallas_call(
        _tc_final_body,
        grid=(NGRID,),
        in_specs=[_blk((NB, D)), _blk((NB, D)), _blk((NB, D)),
                  _full((NP,)), _full((NP,)), _full((D,)),
                  _full((D, ODIM)), _full((ODIM,))],
        out_specs=_blk((NB, ODIM)),
        out_shape=jax.ShapeDtypeStruct((NP, ODIM), jnp.float32),
    )(s0, s1, hs, d0, d1, b2, Wout, bout)


def kernel(x, edge_index, edge_weight, W1, b1, W2, b2, Wout, bout):
    x = x.astype(jnp.float32)
    row = edge_index[0].astype(jnp.int32)
    col = edge_index[1].astype(jnp.int32)
    ew = edge_weight.astype(jnp.float32)

    pad = EP - E
    row2d = jnp.pad(row, (0, pad)).reshape(EP // D, D)
    col2d = jnp.pad(col, (0, pad)).reshape(EP // D, D)
    ew2d = jnp.pad(ew, (0, pad)).reshape(EP // D, D)

    xp = jnp.pad(x, ((0, NP - N), (0, 0)))
    zeros = jnp.zeros((NP, D), jnp.float32)

    d0, d1 = _sc_degree(col2d, ew2d)

    hs1 = _tc_prep(xp, W1, d0, d1)
    s1 = _sc_spmm(hs1, row2d, col2d, ew2d, zeros)
    hs2 = _tc_mid(s1[0], s1[1], hs1, d0, d1, W2, b1)
    s2 = _sc_spmm(hs2, row2d, col2d, ew2d, zeros)
    out = _tc_final(s2[0], s2[1], hs2, d0, d1, b2, Wout, bout)
    return out[:N]
